# baseline (speedup 1.0000x reference)
import functools

import jax
import jax.numpy as jnp
from jax import lax
from jax.experimental import pallas as pl
from jax.experimental.pallas import tpu as pltpu
from jax.experimental.pallas import tpu_sc as plsc

NC, NS, L = 2, 16, 16
NW = NC * NS
ROWS, COLS = 16384, 360
RPW = ROWS // NW
CC = 128
NCH = RPW // CC
GROUPS = CC // L
OCTS = COLS // 8

_mesh = plsc.VectorSubcoreMesh(core_axis_name="c", subcore_axis_name="s")


@functools.partial(
    pl.kernel,
    mesh=_mesh,
    compiler_params=pltpu.CompilerParams(needs_layout_passes=False,
                                         use_tc_tiling_on_sc=True),
    out_type=jax.ShapeDtypeStruct((ROWS,), jnp.float32),
    scratch_types=[
        pltpu.VMEM((COLS, CC), jnp.float32),
        pltpu.VMEM((COLS, CC), jnp.float32),
        pltpu.VMEM((COLS,), jnp.int32),
        pltpu.VMEM((RPW,), jnp.float32),
        pltpu.SemaphoreType.DMA,
        pltpu.SemaphoreType.DMA,
    ],
)
def _argmax_deg_kernel(xt_hbm, degs_hbm, out_hbm, buf0, buf1, degs_v, out_v,
                       sem0, sem1):
    wid = lax.axis_index("s") * NC + lax.axis_index("c")
    col_base = wid * RPW

    pltpu.sync_copy(degs_hbm, degs_v)

    iota = lax.iota(jnp.int32, L)
    neg_inf = jnp.full((L,), -jnp.inf, jnp.float32)
    zero = jnp.zeros((L,), jnp.int32)
    consts = [jnp.full((L,), v, jnp.int32) for v in range(8)]

    bufs = [buf0, buf1]
    sems = [sem0, sem1]
    copies = [None, None]

    def start(ci, b):
        src = xt_hbm.at[:, pl.ds(col_base + ci * CC, CC)]
        copies[b] = pltpu.async_copy(src, bufs[b], sems[b])

    start(0, 0)
    for ci in range(NCH):
        b = ci & 1
        if ci + 1 < NCH:
            start(ci + 1, 1 - b)
        copies[b].wait()
        buf = bufs[b]

        for gp in range(GROUPS // 2):
            c0 = [(2 * gp) * L, (2 * gp + 1) * L]

            def oct_body(o, carry):
                b0, o0, b1, o1 = carry
                r = 8 * o
                ov = jnp.full((L,), o, jnp.int32)
                m = []
                for k in range(2):
                    v = [buf[r + j, pl.ds(c0[k], L)] for j in range(8)]
                    t = [jnp.maximum(v[2 * j], v[2 * j + 1]) for j in range(4)]
                    m.append(jnp.maximum(jnp.maximum(t[0], t[1]),
                                         jnp.maximum(t[2], t[3])))
                p0 = m[0] > b0
                p1 = m[1] > b1
                b0 = jnp.maximum(m[0], b0)
                b1 = jnp.maximum(m[1], b1)
                o0 = jnp.where(p0, ov, o0)
                o1 = jnp.where(p1, ov, o1)
                return b0, o0, b1, o1

            b0, o0, b1, o1 = lax.fori_loop(
                0, OCTS, oct_body, (neg_inf, zero, neg_inf, zero))

            for k, (best, bo) in enumerate(((b0, o0), (b1, o1))):
                r0 = bo * 8
                lanes = c0[k] + iota
                v = [plsc.load_gather(buf, [r0 + consts[j], lanes])
                     for j in range(7)]
                off = consts[7]
                for j in range(6, -1, -1):
                    off = jnp.where(v[j] == best, consts[j], off)
                row = r0 + off
                d = plsc.load_gather(degs_v, [row])
                out_v[pl.ds(ci * CC + c0[k], L)] = d.astype(jnp.float32)

    pltpu.sync_copy(out_v, out_hbm.at[pl.ds(col_base, RPW)])


@jax.jit
def kernel(inputs, degs):
    return _argmax_deg_kernel(inputs.T, degs)
